# baseline (speedup 1.0000x reference)
import functools

import jax
import jax.numpy as jnp
from jax import lax
from jax.experimental import pallas as pl
from jax.experimental.pallas import tpu as pltpu
from jax.experimental.pallas import tpu_sc as plsc

D_MODEL = 512
MAX_LEN = 100
BATCH = 1024
TOKENS = BATCH * MAX_LEN

NC = 2
NS = 16
L = 16
NW = NC * NS
SPAN = TOKENS // NW
CHUNK = 128
NCHUNK = SPAN // CHUNK


def _pos_encoding():
    even_i = jnp.arange(0, D_MODEL, 2, dtype=jnp.float32)
    denominator = jnp.power(10000.0, even_i / D_MODEL)
    position = jnp.arange(MAX_LEN, dtype=jnp.float32).reshape(MAX_LEN, 1)
    even_pe = jnp.sin(position / denominator)
    odd_pe = jnp.cos(position / denominator)
    return jnp.stack([even_pe, odd_pe], axis=2).reshape(MAX_LEN, D_MODEL)


def _make_kernel():
    mesh = plsc.VectorSubcoreMesh(core_axis_name="c", subcore_axis_name="s")

    @functools.partial(
        pl.kernel,
        mesh=mesh,
        out_type=jax.ShapeDtypeStruct((TOKENS, D_MODEL), jnp.float32),
        scratch_types=[
            pltpu.VMEM((SPAN,), jnp.int32),
            pltpu.VMEM((MAX_LEN, D_MODEL), jnp.float32),
            pltpu.VMEM((CHUNK, D_MODEL), jnp.float32),
            pltpu.SemaphoreType.DMA,
        ],
    )
    def k(idx_hbm, table_hbm, pe_hbm, out_hbm, idx_v, pe_v, rows_v, sem):
        wid = lax.axis_index("s") * NC + lax.axis_index("c")
        base = pl.multiple_of(wid * SPAN, SPAN)
        pltpu.sync_copy(pe_hbm, pe_v)
        pltpu.sync_copy(idx_hbm.at[pl.ds(base, SPAN)], idx_v)

        def per_chunk(kc, carry):
            off = kc * CHUNK
            pltpu.async_copy(
                table_hbm.at[idx_v.at[pl.ds(off, CHUNK)]], rows_v, sem
            ).wait()
            pos0 = lax.rem(off, MAX_LEN)

            def add_row(r, carry2):
                p = lax.rem(pos0 + r, MAX_LEN)
                for c in range(D_MODEL // L):
                    sl = pl.ds(c * L, L)
                    rows_v[r, sl] = rows_v[r, sl] + pe_v[p, sl]
                return carry2

            lax.fori_loop(0, CHUNK, add_row, None)
            pltpu.sync_copy(rows_v, out_hbm.at[pl.ds(base + off, CHUNK)])
            return carry

        lax.fori_loop(0, NCHUNK, per_chunk, None)

    return k


def kernel(indices, table):
    pe = _pos_encoding()
    out = _make_kernel()(indices.reshape(TOKENS).astype(jnp.int32), table, pe)
    return out.reshape(BATCH, MAX_LEN, D_MODEL)


# speedup vs baseline: 3.9753x; 3.9753x over previous
import functools

import jax
import jax.numpy as jnp
from jax import lax
from jax.experimental import pallas as pl
from jax.experimental.pallas import tpu as pltpu
from jax.experimental.pallas import tpu_sc as plsc

D_MODEL = 512
MAX_LEN = 100
BATCH = 1024
TOKENS = BATCH * MAX_LEN

NC = 2
NS = 16
L = 16
NW = NC * NS
CHUNK = 128
NBAND = BATCH // CHUNK
L_PER_W = MAX_LEN // (NW // NBAND)


def _pos_encoding():
    even_i = jnp.arange(0, D_MODEL, 2, dtype=jnp.float32)
    denominator = jnp.power(10000.0, even_i / D_MODEL)
    position = jnp.arange(MAX_LEN, dtype=jnp.float32).reshape(MAX_LEN, 1)
    even_pe = jnp.sin(position / denominator)
    odd_pe = jnp.cos(position / denominator)
    return jnp.stack([even_pe, odd_pe], axis=2).reshape(MAX_LEN, D_MODEL)


def _make_kernel():
    mesh = plsc.VectorSubcoreMesh(core_axis_name="c", subcore_axis_name="s")

    @functools.partial(
        pl.kernel,
        mesh=mesh,
        out_type=jax.ShapeDtypeStruct((TOKENS, D_MODEL), jnp.float32),
        scratch_types=[
            pltpu.VMEM((L_PER_W * CHUNK,), jnp.int32),
            pltpu.VMEM((MAX_LEN, D_MODEL), jnp.float32),
            pltpu.VMEM((CHUNK, D_MODEL), jnp.float32),
            pltpu.SemaphoreType.DMA,
            pltpu.SemaphoreType.DMA,
        ],
    )
    def k(idx_hbm, table_hbm, pe_hbm, out_hbm, idx_v, pe_v, rows_v, isem, sem):
        wid = lax.axis_index("s") * NC + lax.axis_index("c")
        lbase = (wid // NBAND) * L_PER_W
        b0 = lax.rem(wid, NBAND) * CHUNK
        pltpu.sync_copy(pe_hbm, pe_v)
        copies = [
            pltpu.async_copy(
                idx_hbm.at[pl.ds((lbase + c) * BATCH + b0, CHUNK)],
                idx_v.at[pl.ds(c * CHUNK, CHUNK)],
                isem,
            )
            for c in range(L_PER_W)
        ]
        for cp in copies:
            cp.wait()

        def per_chunk(c, carry):
            l = lbase + c
            pltpu.async_copy(
                table_hbm.at[idx_v.at[pl.ds(c * CHUNK, CHUNK)]], rows_v, sem
            ).wait()
            pe_regs = tuple(pe_v[l, pl.ds(cc * L, L)] for cc in range(D_MODEL // L))

            def add_row(r, regs):
                for cc in range(D_MODEL // L):
                    sl = pl.ds(cc * L, L)
                    rows_v[r, sl] = rows_v[r, sl] + regs[cc]
                return regs

            lax.fori_loop(0, CHUNK, add_row, pe_regs)
            pltpu.sync_copy(rows_v, out_hbm.at[pl.ds(l * BATCH + b0, CHUNK)])
            return carry

        lax.fori_loop(0, L_PER_W, per_chunk, None)

    return k


def kernel(indices, table):
    pe = _pos_encoding()
    idx_pm = indices.T.reshape(TOKENS).astype(jnp.int32)
    out = _make_kernel()(idx_pm, table, pe)
    return out.reshape(MAX_LEN, BATCH, D_MODEL).transpose(1, 0, 2)


# speedup vs baseline: 5.4936x; 1.3819x over previous
import functools

import jax
import jax.numpy as jnp
from jax import lax
from jax.experimental import pallas as pl
from jax.experimental.pallas import tpu as pltpu
from jax.experimental.pallas import tpu_sc as plsc

D_MODEL = 512
MAX_LEN = 100
BATCH = 1024
TOKENS = BATCH * MAX_LEN

NC = 2
NS = 16
L = 16
NW = NC * NS
CHUNK = 64
NBAND = BATCH // CHUNK
L_PER_W = MAX_LEN // (NW // NBAND)
NPAIR = L_PER_W // 2

VLANES = D_MODEL // L


def _pos_encoding():
    even_i = jnp.arange(0, D_MODEL, 2, dtype=jnp.float32)
    denominator = jnp.power(10000.0, even_i / D_MODEL)
    position = jnp.arange(MAX_LEN, dtype=jnp.float32).reshape(MAX_LEN, 1)
    even_pe = jnp.sin(position / denominator)
    odd_pe = jnp.cos(position / denominator)
    return jnp.stack([even_pe, odd_pe], axis=2).reshape(MAX_LEN, D_MODEL)


def _make_kernel():
    mesh = plsc.VectorSubcoreMesh(core_axis_name="c", subcore_axis_name="s")

    @functools.partial(
        pl.kernel,
        mesh=mesh,
        out_type=jax.ShapeDtypeStruct((TOKENS, D_MODEL), jnp.float32),
        scratch_types=[
            pltpu.VMEM((L_PER_W * CHUNK,), jnp.int32),
            pltpu.VMEM((MAX_LEN, D_MODEL), jnp.float32),
            pltpu.VMEM((CHUNK, D_MODEL), jnp.float32),
            pltpu.VMEM((CHUNK, D_MODEL), jnp.float32),
            pltpu.SemaphoreType.DMA,
            pltpu.SemaphoreType.DMA,
            pltpu.SemaphoreType.DMA,
        ],
    )
    def k(idx_hbm, table_hbm, pe_hbm, out_hbm, idx_v, pe_v, rows0, rows1,
          isem, gsem, wsem):
        wid = lax.axis_index("s") * NC + lax.axis_index("c")
        lbase = (wid // NBAND) * L_PER_W
        b0 = lax.rem(wid, NBAND) * CHUNK
        pltpu.sync_copy(pe_hbm, pe_v)
        copies = [
            pltpu.async_copy(
                idx_hbm.at[pl.ds((lbase + c) * BATCH + b0, CHUNK)],
                idx_v.at[pl.ds(c * CHUNK, CHUNK)],
                isem,
            )
            for c in range(L_PER_W)
        ]
        for cp in copies:
            cp.wait()

        def gather(c, buf):
            pltpu.async_copy(
                table_hbm.at[idx_v.at[pl.ds(c * CHUNK, CHUNK)]], buf, gsem
            )

        def drain_gather(buf):
            pltpu.make_async_copy(table_hbm.at[pl.ds(0, CHUNK)], buf, gsem).wait()

        def store(c, buf):
            pltpu.async_copy(
                buf, out_hbm.at[pl.ds((lbase + c) * BATCH + b0, CHUNK)], wsem
            )

        def drain_store(buf):
            pltpu.make_async_copy(buf, out_hbm.at[pl.ds(0, CHUNK)], wsem).wait()

        def add_pe(c, buf):
            pe_regs = tuple(
                pe_v[lbase + c, pl.ds(cc * L, L)] for cc in range(VLANES)
            )

            def add_row(r, regs):
                for cc in range(VLANES):
                    sl = pl.ds(cc * L, L)
                    buf[r, sl] = buf[r, sl] + regs[cc]
                return regs

            lax.fori_loop(0, CHUNK, add_row, pe_regs)

        gather(0, rows0)

        def body(j, carry):
            c0 = 2 * j
            drain_gather(rows0)

            @pl.when(j > 0)
            def _():
                drain_store(rows1)

            gather(c0 + 1, rows1)
            add_pe(c0, rows0)
            store(c0, rows0)

            drain_gather(rows1)
            drain_store(rows0)

            @pl.when(j < NPAIR - 1)
            def _():
                gather(c0 + 2, rows0)

            add_pe(c0 + 1, rows1)
            store(c0 + 1, rows1)
            return carry

        lax.fori_loop(0, NPAIR, body, None)
        drain_store(rows1)

    return k


def kernel(indices, table):
    pe = _pos_encoding()
    idx_pm = indices.T.reshape(TOKENS).astype(jnp.int32)
    out = _make_kernel()(idx_pm, table, pe)
    return out.reshape(MAX_LEN, BATCH, D_MODEL).transpose(1, 0, 2)


# speedup vs baseline: 5.6989x; 1.0374x over previous
import functools

import jax
import jax.numpy as jnp
import numpy as np
from jax import lax
from jax.experimental import pallas as pl
from jax.experimental.pallas import tpu as pltpu
from jax.experimental.pallas import tpu_sc as plsc

D_MODEL = 512
MAX_LEN = 100
BATCH = 1024
TOKENS = BATCH * MAX_LEN

NC = 2
NS = 16
L = 16
NW = NC * NS
CHUNK = BATCH // NW
NBUF = 4
AHEAD = 2
VLANES = D_MODEL // L


def _pos_encoding_np():
    even_i = np.arange(0, D_MODEL, 2, dtype=np.float32)
    denominator = np.power(10000.0, even_i / D_MODEL)
    position = np.arange(MAX_LEN, dtype=np.float32).reshape(MAX_LEN, 1)
    even_pe = np.sin(position / denominator)
    odd_pe = np.cos(position / denominator)
    return np.stack([even_pe, odd_pe], axis=2).reshape(MAX_LEN, D_MODEL)


_PE = _pos_encoding_np()


def _make_kernel():
    mesh = plsc.VectorSubcoreMesh(core_axis_name="c", subcore_axis_name="s")

    @functools.partial(
        pl.kernel,
        mesh=mesh,
        out_type=jax.ShapeDtypeStruct((TOKENS, D_MODEL), jnp.float32),
        scratch_types=[
            pltpu.VMEM((MAX_LEN * CHUNK,), jnp.int32),
            pltpu.VMEM((MAX_LEN, D_MODEL), jnp.float32),
        ]
        + [pltpu.VMEM((CHUNK, D_MODEL), jnp.float32) for _ in range(NBUF)]
        + [pltpu.SemaphoreType.DMA, pltpu.SemaphoreType.DMA],
    )
    def k(idx_hbm, table_hbm, pe_hbm, out_hbm, idx_v, pe_v,
          rows0, rows1, rows2, rows3, gsem, wsem):
        bufs = (rows0, rows1, rows2, rows3)
        wid = lax.axis_index("s") * NC + lax.axis_index("c")
        b0 = wid * CHUNK
        pltpu.sync_copy(pe_hbm, pe_v)
        pltpu.sync_copy(idx_hbm.at[pl.ds(wid * MAX_LEN * CHUNK, MAX_LEN * CHUNK)],
                        idx_v)

        def gather(c, buf):
            pltpu.async_copy(
                table_hbm.at[idx_v.at[pl.ds(c * CHUNK, CHUNK)]], buf, gsem
            )

        def drain_gather(buf):
            pltpu.make_async_copy(table_hbm.at[pl.ds(0, CHUNK)], buf, gsem).wait()

        def store(c, buf):
            pltpu.async_copy(
                buf, out_hbm.at[pl.ds(c * BATCH + b0, CHUNK)], wsem
            )

        def drain_store(buf):
            pltpu.make_async_copy(buf, out_hbm.at[pl.ds(0, CHUNK)], wsem).wait()

        def add_pe(c, buf):
            pe_regs = tuple(pe_v[c, pl.ds(cc * L, L)] for cc in range(VLANES))

            def add_row(r, regs):
                for cc in range(VLANES):
                    sl = pl.ds(cc * L, L)
                    buf[r, sl] = buf[r, sl] + regs[cc]
                return regs

            lax.fori_loop(0, CHUNK, add_row, pe_regs)

        for c in range(AHEAD):
            gather(c, bufs[c])

        def body(j, carry):
            for kk in range(NBUF):
                c = NBUF * j + kk
                drain_gather(bufs[kk])

                @pl.when(c >= AHEAD)
                def _():
                    drain_store(bufs[(kk + AHEAD) % NBUF])

                @pl.when(c < MAX_LEN - AHEAD)
                def _():
                    gather(c + AHEAD, bufs[(kk + AHEAD) % NBUF])

                add_pe(c, bufs[kk])
                store(c, bufs[kk])
            return carry

        lax.fori_loop(0, MAX_LEN // NBUF, body, None)
        for kk in range(AHEAD):
            drain_store(bufs[(MAX_LEN - AHEAD + kk) % NBUF])

    return k


def kernel(indices, table):
    idx_r = (indices.astype(jnp.int32)
             .T.reshape(MAX_LEN, NW, CHUNK)
             .transpose(1, 0, 2)
             .reshape(TOKENS))
    pe = jnp.asarray(_PE)
    out = _make_kernel()(idx_r, table, pe)
    return out.reshape(MAX_LEN, BATCH, D_MODEL).transpose(1, 0, 2)
